# baseline (speedup 1.0000x reference)
import jax
import jax.numpy as jnp
from jax import lax
from jax.experimental import pallas as pl
from jax.experimental.pallas import tpu as pltpu


def _scores_kernel(x_ref, w_ref, b_ref, s_ref):
    x = x_ref[...].astype(jnp.float32)
    w = w_ref[...].astype(jnp.float32)
    logits = lax.dot_general(w, x, (((1,), (1,)), ((), ())),
                             preferred_element_type=jnp.float32)
    s_ref[...] = jax.nn.sigmoid((logits + b_ref[0, 0]) * 0.01)


def _make_pool_kernel(tr, K, N, D):
    KQ = K // 2

    def body(idx_sm,
             b8_sm,
             sh_sm,
             a_hbm,
             x_vmem,
             val_ref,
             newa_hbm, newx_ref,
             stage, stage_t, g_buf,
             ob0, ob1,
             sem, osems):
        t = pl.program_id(0)
        nt = pl.num_programs(0)
        base = t * tr

        iota8x = lax.broadcasted_iota(jnp.int32, (8, D), 0)

        def make_xgather(issue_rows):
            def xgather(g, carry):
                r0 = g * 16
                if issue_rows:
                    for r in range(16):
                        src = idx_sm[r0 + r]
                        pltpu.make_async_copy(a_hbm.at[pl.ds(src, 1), :],
                                              stage.at[pl.ds(r0 + r, 1), :],
                                              sem).start()
                for grp in range(2):
                    j0 = pl.multiple_of(r0 + grp * 8, 8)
                    acc = jnp.zeros((8, D), jnp.float32)
                    for s in range(8):
                        j = base + r0 + grp * 8 + s
                        c8 = pl.multiple_of(b8_sm[j], 8)
                        chunk = x_vmem[pl.ds(c8, 8), :]
                        shifted = pltpu.roll(chunk, sh_sm[j], axis=0)
                        acc = jnp.where(iota8x == s, shifted, acc)
                    newx_ref[pl.ds(j0, 8), :] = acc * val_ref[pl.ds(j0, 8), :]
                return carry
            return xgather

        @pl.when(t == 0)
        def _first():
            lax.fori_loop(0, tr // 16, make_xgather(True), 0)

        @pl.when(t > 0)
        def _later():
            lax.fori_loop(0, tr // 16, make_xgather(False), 0)

        pltpu.make_async_copy(a_hbm.at[pl.ds(0, tr), :], stage.at[...],
                              sem).wait()

        for c in range(N // 128):
            blk = stage[:, c * 128:(c + 1) * 128]
            stage_t[c * 128:(c + 1) * 128, :] = jnp.transpose(blk, (1, 0))

        @pl.when(t + 1 < nt)
        def _issue_next():
            nbase = base + tr

            def issue(g, carry):
                r0 = g * 16
                for r in range(16):
                    src = idx_sm[nbase + r0 + r]
                    pltpu.make_async_copy(a_hbm.at[pl.ds(src, 1), :],
                                          stage.at[pl.ds(r0 + r, 1), :],
                                          sem).start()
                return carry
            lax.fori_loop(0, tr // 16, issue, 0)

        iota8a = lax.broadcasted_iota(jnp.int32, (8, tr), 0)

        def make_agather(quarter):
            jbase = quarter * KQ

            def agather(g, carry):
                for grp in range(16):
                    j0 = pl.multiple_of(g * 128 + grp * 8, 8)
                    acc = jnp.zeros((8, tr), jnp.float32)
                    for s in range(8):
                        j = jbase + g * 128 + grp * 8 + s
                        c8 = pl.multiple_of(b8_sm[j], 8)
                        chunk = stage_t[pl.ds(c8, 8), :]
                        shifted = pltpu.roll(chunk, sh_sm[j], axis=0)
                        acc = jnp.where(iota8a == s, shifted, acc)
                    g_buf[pl.ds(j0, 8), :] = acc
                return carry
            return agather

        obufs = (ob0, ob1)
        for quarter in range(2):
            lax.fori_loop(0, KQ // 128, make_agather(quarter), 0)
            jbase = quarter * KQ
            ob = obufs[quarter & 1]

            def _wait_ob(ob=ob, b=quarter & 1):
                pltpu.make_async_copy(
                    ob.at[...], newa_hbm.at[pl.ds(0, tr), pl.ds(0, KQ)],
                    osems.at[b]).wait()
            pl.when(t > 0)(_wait_ob)
            for c in range(KQ // 128):
                blk = g_buf[c * 128:(c + 1) * 128, :]
                ob[:, c * 128:(c + 1) * 128] = jnp.transpose(blk, (1, 0))
            pltpu.make_async_copy(
                ob.at[...],
                newa_hbm.at[pl.ds(base, tr), pl.ds(jbase, KQ)],
                osems.at[quarter & 1]).start()

        @pl.when(t + 1 == nt)
        def _drain_outputs():
            for b, ob in enumerate(obufs):
                pltpu.make_async_copy(
                    ob.at[...], newa_hbm.at[pl.ds(0, tr), pl.ds(0, KQ)],
                    osems.at[b]).wait()

    return body


def kernel(A, X, weight, bias):
    N = A.shape[0]
    D = X.shape[1]
    K = int(0.5 * N)

    w2d = weight.reshape(1, D).astype(jnp.float32)
    b2d = bias.reshape(1, 1).astype(jnp.float32)

    scores = pl.pallas_call(
        _scores_kernel,
        out_shape=jax.ShapeDtypeStruct((1, N), jnp.float32),
        grid=(1,),
        in_specs=[
            pl.BlockSpec((N, D), lambda i: (0, 0)),
            pl.BlockSpec((1, D), lambda i: (0, 0)),
            pl.BlockSpec(memory_space=pltpu.MemorySpace.SMEM),
        ],
        out_specs=pl.BlockSpec((1, N), lambda i: (0, 0)),
        compiler_params=pltpu.CompilerParams(dimension_semantics=("parallel",)),
    )(X, w2d, b2d)

    values, idx = lax.top_k(scores[0], K)
    idx = idx.astype(jnp.int32)

    TR = min(1024, K)
    while K % TR:
        TR //= 2

    b8 = jnp.bitwise_and(idx, -8)
    shifts = jnp.bitwise_and(
        jnp.bitwise_and(jnp.arange(K, dtype=jnp.int32), 7)
        - jnp.bitwise_and(idx, 7), 7)

    grid_spec = pltpu.PrefetchScalarGridSpec(
        num_scalar_prefetch=3,
        grid=(K // TR,),
        in_specs=[
            pl.BlockSpec(memory_space=pl.ANY),
            pl.BlockSpec((N, D), lambda t, i, i2, i3: (0, 0)),
            pl.BlockSpec((TR, 1), lambda t, i, i2, i3: (t, 0)),
        ],
        out_specs=(
            pl.BlockSpec(memory_space=pl.ANY),
            pl.BlockSpec((TR, D), lambda t, i, i2, i3: (t, 0)),
        ),
        scratch_shapes=[
            pltpu.VMEM((TR, N), jnp.float32),
            pltpu.VMEM((N, TR), jnp.float32),
            pltpu.VMEM((K // 2, TR), jnp.float32),
            pltpu.VMEM((TR, K // 2), jnp.float32),
            pltpu.VMEM((TR, K // 2), jnp.float32),
            pltpu.SemaphoreType.DMA,
            pltpu.SemaphoreType.DMA((2,)),
        ],
    )
    new_A, new_X = pl.pallas_call(
        _make_pool_kernel(TR, K, N, D),
        out_shape=(
            jax.ShapeDtypeStruct((K, K), A.dtype),
            jax.ShapeDtypeStruct((K, D), X.dtype),
        ),
        grid_spec=grid_spec,
        compiler_params=pltpu.CompilerParams(
            dimension_semantics=("arbitrary",)),
    )(idx, b8, shifts, A, X, values.reshape(K, 1).astype(jnp.float32))

    return new_A, new_X, idx


# speedup vs baseline: 1.0076x; 1.0076x over previous
import jax
import jax.numpy as jnp
from jax import lax
from jax.experimental import pallas as pl
from jax.experimental.pallas import tpu as pltpu


def _scores_kernel(x_ref, w_ref, b_ref, s_ref):
    x = x_ref[...].astype(jnp.float32)
    w = w_ref[...].astype(jnp.float32)
    logits = lax.dot_general(w, x, (((1,), (1,)), ((), ())),
                             preferred_element_type=jnp.float32)
    s_ref[...] = jax.nn.sigmoid((logits + b_ref[0, 0]) * 0.01)


def _make_pool_kernel(tr, K, N, D):
    KQ = K // 4

    def body(idx_sm,
             b8_sm,
             sh_sm,
             a_hbm,
             x_vmem,
             val_ref,
             newa_hbm, newx_ref,
             stage, stage_t, g_buf,
             ob0, ob1,
             sem, osems):
        t = pl.program_id(0)
        nt = pl.num_programs(0)
        base = t * tr

        iota8x = lax.broadcasted_iota(jnp.int32, (8, D), 0)

        def make_xgather(issue_rows):
            def xgather(g, carry):
                r0 = g * 16
                if issue_rows:
                    for r in range(16):
                        src = idx_sm[r0 + r]
                        pltpu.make_async_copy(a_hbm.at[pl.ds(src, 1), :],
                                              stage.at[pl.ds(r0 + r, 1), :],
                                              sem).start()
                for grp in range(2):
                    j0 = pl.multiple_of(r0 + grp * 8, 8)
                    acc = jnp.zeros((8, D), jnp.float32)
                    for s in range(8):
                        j = base + r0 + grp * 8 + s
                        c8 = pl.multiple_of(b8_sm[j], 8)
                        chunk = x_vmem[pl.ds(c8, 8), :]
                        shifted = pltpu.roll(chunk, sh_sm[j], axis=0)
                        acc = jnp.where(iota8x == s, shifted, acc)
                    newx_ref[pl.ds(j0, 8), :] = acc * val_ref[pl.ds(j0, 8), :]
                return carry
            return xgather

        @pl.when(t == 0)
        def _first():
            lax.fori_loop(0, tr // 16, make_xgather(True), 0)

        @pl.when(t > 0)
        def _later():
            lax.fori_loop(0, tr // 16, make_xgather(False), 0)

        pltpu.make_async_copy(a_hbm.at[pl.ds(0, tr), :], stage.at[...],
                              sem).wait()

        for c in range(N // 128):
            blk = stage[:, c * 128:(c + 1) * 128]
            stage_t[c * 128:(c + 1) * 128, :] = jnp.transpose(blk, (1, 0))

        @pl.when(t + 1 < nt)
        def _issue_next():
            nbase = base + tr

            def issue(g, carry):
                r0 = g * 16
                for r in range(16):
                    src = idx_sm[nbase + r0 + r]
                    pltpu.make_async_copy(a_hbm.at[pl.ds(src, 1), :],
                                          stage.at[pl.ds(r0 + r, 1), :],
                                          sem).start()
                return carry
            lax.fori_loop(0, tr // 16, issue, 0)

        iota8a = lax.broadcasted_iota(jnp.int32, (8, tr), 0)

        def make_agather(quarter):
            jbase = quarter * KQ

            def agather(g, carry):
                for grp in range(16):
                    j0 = pl.multiple_of(g * 128 + grp * 8, 8)
                    acc = jnp.zeros((8, tr), jnp.float32)
                    for s in range(8):
                        j = jbase + g * 128 + grp * 8 + s
                        c8 = pl.multiple_of(b8_sm[j], 8)
                        chunk = stage_t[pl.ds(c8, 8), :]
                        shifted = pltpu.roll(chunk, sh_sm[j], axis=0)
                        acc = jnp.where(iota8a == s, shifted, acc)
                    g_buf[pl.ds(j0, 8), :] = acc
                return carry
            return agather

        obufs = (ob0, ob1)
        for quarter in range(4):
            lax.fori_loop(0, KQ // 128, make_agather(quarter), 0)
            jbase = quarter * KQ
            ob = obufs[quarter & 1]

            def _wait_ob(ob=ob, b=quarter & 1):
                pltpu.make_async_copy(
                    ob.at[...], newa_hbm.at[pl.ds(0, tr), pl.ds(0, KQ)],
                    osems.at[b]).wait()
            if quarter < 2:
                pl.when(t > 0)(_wait_ob)
            else:
                _wait_ob()
            for c in range(KQ // 128):
                blk = g_buf[c * 128:(c + 1) * 128, :]
                ob[:, c * 128:(c + 1) * 128] = jnp.transpose(blk, (1, 0))
            pltpu.make_async_copy(
                ob.at[...],
                newa_hbm.at[pl.ds(base, tr), pl.ds(jbase, KQ)],
                osems.at[quarter & 1]).start()

        @pl.when(t + 1 == nt)
        def _drain_outputs():
            for b, ob in enumerate(obufs):
                pltpu.make_async_copy(
                    ob.at[...], newa_hbm.at[pl.ds(0, tr), pl.ds(0, KQ)],
                    osems.at[b]).wait()

    return body


def kernel(A, X, weight, bias):
    N = A.shape[0]
    D = X.shape[1]
    K = int(0.5 * N)

    w2d = weight.reshape(1, D).astype(jnp.float32)
    b2d = bias.reshape(1, 1).astype(jnp.float32)

    scores = pl.pallas_call(
        _scores_kernel,
        out_shape=jax.ShapeDtypeStruct((1, N), jnp.float32),
        grid=(1,),
        in_specs=[
            pl.BlockSpec((N, D), lambda i: (0, 0)),
            pl.BlockSpec((1, D), lambda i: (0, 0)),
            pl.BlockSpec(memory_space=pltpu.MemorySpace.SMEM),
        ],
        out_specs=pl.BlockSpec((1, N), lambda i: (0, 0)),
        compiler_params=pltpu.CompilerParams(dimension_semantics=("parallel",)),
    )(X, w2d, b2d)

    values, idx = lax.top_k(scores[0], K)
    idx = idx.astype(jnp.int32)

    TR = min(1024, K)
    while K % TR:
        TR //= 2

    b8 = jnp.bitwise_and(idx, -8)
    shifts = jnp.bitwise_and(
        jnp.bitwise_and(jnp.arange(K, dtype=jnp.int32), 7)
        - jnp.bitwise_and(idx, 7), 7)

    grid_spec = pltpu.PrefetchScalarGridSpec(
        num_scalar_prefetch=3,
        grid=(K // TR,),
        in_specs=[
            pl.BlockSpec(memory_space=pl.ANY),
            pl.BlockSpec((N, D), lambda t, i, i2, i3: (0, 0)),
            pl.BlockSpec((TR, 1), lambda t, i, i2, i3: (t, 0)),
        ],
        out_specs=(
            pl.BlockSpec(memory_space=pl.ANY),
            pl.BlockSpec((TR, D), lambda t, i, i2, i3: (t, 0)),
        ),
        scratch_shapes=[
            pltpu.VMEM((TR, N), jnp.float32),
            pltpu.VMEM((N, TR), jnp.float32),
            pltpu.VMEM((K // 4, TR), jnp.float32),
            pltpu.VMEM((TR, K // 4), jnp.float32),
            pltpu.VMEM((TR, K // 4), jnp.float32),
            pltpu.SemaphoreType.DMA,
            pltpu.SemaphoreType.DMA((2,)),
        ],
    )
    new_A, new_X = pl.pallas_call(
        _make_pool_kernel(TR, K, N, D),
        out_shape=(
            jax.ShapeDtypeStruct((K, K), A.dtype),
            jax.ShapeDtypeStruct((K, D), X.dtype),
        ),
        grid_spec=grid_spec,
        compiler_params=pltpu.CompilerParams(
            dimension_semantics=("arbitrary",)),
    )(idx, b8, shifts, A, X, values.reshape(K, 1).astype(jnp.float32))

    return new_A, new_X, idx
